# baseline (speedup 1.0000x reference)
import jax
import jax.numpy as jnp
from jax.experimental import pallas as pl

_BS = 512


def _add_body(x_ref, p_ref, o_ref):
    o_ref[...] = x_ref[...] + p_ref[...]


def kernel(x, pos_table):
    B, T, D = x.shape
    pe = pos_table[:T]
    n_seq = T // _BS
    return pl.pallas_call(
        _add_body,
        grid=(n_seq,),
        in_specs=[
            pl.BlockSpec((B, _BS, D), lambda s: (0, s, 0)),
            pl.BlockSpec((_BS, D), lambda s: (s, 0)),
        ],
        out_specs=pl.BlockSpec((B, _BS, D), lambda s: (0, s, 0)),
        out_shape=jax.ShapeDtypeStruct((B, T, D), x.dtype),
    )(x, pe)


# speedup vs baseline: 1.0055x; 1.0055x over previous
import jax
import jax.numpy as jnp
from jax.experimental import pallas as pl

_BS = 2048


def _add_body(x_ref, p_ref, o_ref):
    o_ref[...] = x_ref[...] + p_ref[...]


def kernel(x, pos_table):
    B, T, D = x.shape
    pe = pos_table[:T]
    n_seq = T // _BS
    return pl.pallas_call(
        _add_body,
        grid=(n_seq, B),
        in_specs=[
            pl.BlockSpec((1, _BS, D), lambda s, b: (b, s, 0)),
            pl.BlockSpec((_BS, D), lambda s, b: (s, 0)),
        ],
        out_specs=pl.BlockSpec((1, _BS, D), lambda s, b: (b, s, 0)),
        out_shape=jax.ShapeDtypeStruct((B, T, D), x.dtype),
    )(x, pe)
